# Initial kernel scaffold; baseline (speedup 1.0000x reference)
import jax
import jax.numpy as jnp
from jax.experimental import pallas as pl


def kernel(roles, contexts, agent_num, init_role_embedding, W1, b1, W21, b21, W22, b22, W3, b3, W4, b4, Wc, bc):
    raise NotImplementedError("write your pallas kernel here")


# speedup vs baseline: 1.7835x; 1.7835x over previous
import functools
import math

import jax
import jax.numpy as jnp
from jax.experimental import pallas as pl
from jax.experimental.pallas import tpu as pltpu

_STD2 = 0.1
_VAR2 = _STD2 * _STD2
_LOG_VAR2 = math.log(_VAR2)

B, R, D, C, H = 16, 4096, 384, 128, 64
RB = 1024
MAX_AGENTS = 4


def _dot(a, b):
    return jax.lax.dot_general(
        a, b, (((1,), (0,)), ((), ())),
        precision=jax.lax.Precision.HIGHEST,
        preferred_element_type=jnp.float32)


def _vae_body(roles_ref, eps_ref, W1_ref, b1_ref, W21_ref, b21_ref,
              W22_ref, b22_ref, W3_ref, b3_ref, W4_ref, b4_ref,
              re_ref, acc_ref):
    rb = pl.program_id(1)
    x = roles_ref[0]
    eps = eps_ref[0]
    h = jnp.maximum(_dot(x, W1_ref[...]) + b1_ref[...], 0.0)
    mu = _dot(h, W21_ref[...]) + b21_ref[...]
    lv = _dot(h, W22_ref[...]) + b22_ref[...]
    std = jnp.exp(0.5 * lv) * _STD2
    z = mu + eps * std
    dec = jnp.maximum(_dot(z, W3_ref[...]) + b3_ref[...], 0.0)
    x_hat = _dot(dec, W4_ref[...]) + b4_ref[...]
    ss = jnp.sum(z * z, axis=1, keepdims=True)
    re = z / jnp.maximum(jnp.sqrt(ss), 1e-12)
    re_ref[0] = re
    mse_p = jnp.sum((x_hat - x) ** 2)
    kld_p = jnp.sum(1.0 - _LOG_VAR2 + lv - (mu * mu + jnp.exp(lv)) / _VAR2)
    contrib = mse_p / (R * D) - 0.5 * kld_p / (R * H)

    @pl.when(rb == 0)
    def _():
        acc_ref[...] = jnp.zeros_like(acc_ref)

    acc_ref[...] += contrib


def _samp_body(re_ref, wc_ref, bc_ref, aux_ref, lp_ref, sum_ref):
    aux = aux_ref[0]
    ctx_row = aux[0:1, :]
    re = re_ref[0]
    hist = aux[3:4, 0:H]
    cur = aux[3:4, 0:H]
    lp = jnp.float32(0.0)
    row_iota = jax.lax.broadcasted_iota(jnp.int32, (R, 1), 0)
    lane_iota = jax.lax.broadcasted_iota(jnp.int32, (1, R), 1)
    for j in range(MAX_AGENTS):
        r_j = aux[1, j]
        act = aux[2, j]
        hc = hist + cur
        m = jnp.mean(hc, axis=1, keepdims=True)
        v = jnp.mean((hc - m) ** 2, axis=1, keepdims=True)
        hist_new = (hc - m) / jnp.sqrt(v + 1e-5)
        cat = jnp.concatenate([ctx_row, hist_new], axis=1)
        ctxv = _dot(cat, wc_ref[...]) + bc_ref[...]
        ctxv = ctxv / jnp.maximum(jnp.sqrt(jnp.sum(ctxv * ctxv)), 1e-12)
        logits = jax.lax.dot_general(
            ctxv, re, (((1,), (1,)), ((), ())),
            precision=jax.lax.Precision.HIGHEST,
            preferred_element_type=jnp.float32)
        p = jnp.exp(logits - jnp.max(logits))
        scores = p / jnp.sum(p)
        cs = scores
        k = 1
        while k < R:
            shifted = jnp.concatenate(
                [jnp.zeros((1, k), dtype=jnp.float32), cs[:, :R - k]], axis=1)
            cs = cs + shifted
            k *= 2
        cnt = jnp.sum((cs <= r_j).astype(jnp.int32))
        sel = jnp.where(cnt >= R, 0, cnt)
        psel = jnp.sum(jnp.where(lane_iota == sel, scores, 0.0))
        currow = jnp.sum(jnp.where(row_iota == sel, re, 0.0),
                         axis=0, keepdims=True)
        lp = lp + act * jnp.log(psel)
        hist = hist * (1.0 - act) + hist_new * act
        cur = cur * (1.0 - act) + currow * act
    lp_ref[...] = jnp.full((1, 1, 128), lp, dtype=jnp.float32)
    sum_ref[...] = jnp.concatenate(
        [hist, jnp.zeros((1, 128 - H), dtype=jnp.float32)], axis=1).reshape(1, 1, 128)


def kernel(roles, contexts, agent_num, init_role_embedding,
           W1, b1, W21, b21, W22, b22, W3, b3, W4, b4, Wc, bc):
    f32 = jnp.float32
    eps_keys = jax.vmap(
        lambda i: jax.random.fold_in(jax.random.key(42), i))(jnp.arange(B))
    eps = jax.vmap(
        lambda k: jax.random.normal(k, (R, H), dtype=f32))(eps_keys)
    rcounts = (jnp.arange(B)[:, None] * 100003 + jnp.arange(MAX_AGENTS)[None, :]).reshape(-1)
    rkeys = jax.vmap(
        lambda c: jax.random.fold_in(jax.random.key(7), c))(rcounts)
    rs = jax.vmap(lambda k: jax.random.uniform(k, (1, 1))[0, 0])(rkeys)
    rs = rs.reshape(B, MAX_AGENTS)

    active = (jnp.arange(MAX_AGENTS)[None, :] < agent_num[:, None]).astype(f32)
    aux = jnp.zeros((B, 8, 128), dtype=f32)
    aux = aux.at[:, 0, :C].set(contexts)
    aux = aux.at[:, 1, :MAX_AGENTS].set(rs)
    aux = aux.at[:, 2, :MAX_AGENTS].set(active)
    aux = aux.at[:, 3, :H].set(jnp.broadcast_to(init_role_embedding, (B, H)))

    b1r, b21r, b22r, b3r = (v.reshape(1, H) for v in (b1, b21, b22, b3))
    b4r = b4.reshape(1, D)
    bcr = bc.reshape(1, H)

    wspec = lambda shp: pl.BlockSpec(shp, lambda i, r: (0, 0))
    re_arr, acc = pl.pallas_call(
        _vae_body,
        grid=(B, R // RB),
        in_specs=[
            pl.BlockSpec((1, RB, D), lambda i, r: (i, r, 0)),
            pl.BlockSpec((1, RB, H), lambda i, r: (i, r, 0)),
            wspec((D, H)), wspec((1, H)),
            wspec((H, H)), wspec((1, H)),
            wspec((H, H)), wspec((1, H)),
            wspec((H, H)), wspec((1, H)),
            wspec((H, D)), wspec((1, D)),
        ],
        out_specs=[
            pl.BlockSpec((1, RB, H), lambda i, r: (i, r, 0)),
            pl.BlockSpec((1, 1, 128), lambda i, r: (i, 0, 0)),
        ],
        out_shape=[
            jax.ShapeDtypeStruct((B, R, H), f32),
            jax.ShapeDtypeStruct((B, 1, 128), f32),
        ],
    )(roles, eps, W1, b1r, W21, b21r, W22, b22r, W3, b3r, W4, b4r)

    lp_out, sum_out = pl.pallas_call(
        _samp_body,
        grid=(B,),
        in_specs=[
            pl.BlockSpec((1, R, H), lambda i: (i, 0, 0)),
            pl.BlockSpec((C + H, H), lambda i: (0, 0)),
            pl.BlockSpec((1, H), lambda i: (0, 0)),
            pl.BlockSpec((1, 8, 128), lambda i: (i, 0, 0)),
        ],
        out_specs=[
            pl.BlockSpec((1, 1, 128), lambda i: (i, 0, 0)),
            pl.BlockSpec((1, 1, 128), lambda i: (i, 0, 0)),
        ],
        out_shape=[
            jax.ShapeDtypeStruct((B, 1, 128), f32),
            jax.ShapeDtypeStruct((B, 1, 128), f32),
        ],
    )(re_arr, Wc, bcr, aux)

    log_probs = lp_out[:, 0, :1]
    summary_role = sum_out[:, 0, :H]
    vae_loss = jnp.sum(acc[:, 0, 0]) / B
    return (log_probs, summary_role, vae_loss)
